# Initial kernel scaffold; baseline (speedup 1.0000x reference)
import jax
import jax.numpy as jnp
from jax.experimental import pallas as pl


def kernel(x, W_in, b_in, in_proj_w, in_proj_b, out_proj_w, out_proj_b, W_g, b_g, W1, b1, W2, b2):
    raise NotImplementedError("write your pallas kernel here")


# speedup vs baseline: 2.6822x; 2.6822x over previous
import jax
import jax.numpy as jnp
from jax import lax
from jax.experimental import pallas as pl

N_TOK = 4096
D = 768
E = 8
DH = 256
GLANES = 128
TILE = 512
NEG = -1e30


def _top2(logits):
    iota = lax.broadcasted_iota(jnp.int32, logits.shape, 1)
    m1 = jnp.max(logits, axis=1, keepdims=True)
    i1 = jnp.min(jnp.where(logits == m1, iota, GLANES), axis=1, keepdims=True)
    lm = jnp.where(iota == i1, NEG, logits)
    m2 = jnp.max(lm, axis=1, keepdims=True)
    i2 = jnp.min(jnp.where(lm == m2, iota, GLANES), axis=1, keepdims=True)
    w1 = 1.0 / (1.0 + jnp.exp(m2 - m1))
    w2 = 1.0 - w1
    return i1, i2, w1, w2


def _dg(a, b):
    return lax.dot_general(a, b, (((1,), (1,)), ((), ())),
                           preferred_element_type=jnp.float32)


def _moe_body(x_ref, Wi_ref, bi_ref, Wv_ref, bv_ref, Wo_ref, bo_ref, Wg_ref,
              bg_ref, W1_ref, b1_ref, W2_ref, b2_ref, out_ref):
    f32 = jnp.float32
    x1 = _dg(x_ref[...], Wi_ref[...]) + bi_ref[...]
    v = _dg(x1, Wv_ref[...]) + bv_ref[...]
    x2 = _dg(v, Wo_ref[...]) + bo_ref[...]
    logits = _dg(x2, Wg_ref[...]) + bg_ref[...]
    i1, i2, w1, w2 = _top2(logits)
    acc = jnp.zeros((x2.shape[0], D), f32)
    for e in range(E):
        h = jnp.maximum(_dg(x2, W1_ref[e]) + b1_ref[e], 0.0)
        y = _dg(h, W2_ref[e]) + b2_ref[e]
        we = (w1 * (i1 == e).astype(f32) + w2 * (i2 == e).astype(f32))
        acc = acc + we * y
    out_ref[...] = acc


def kernel(x, W_in, b_in, in_proj_w, in_proj_b, out_proj_w, out_proj_b,
           W_g, b_g, W1, b1, W2, b2):
    Wv = in_proj_w[2 * D:]
    bv = in_proj_b[2 * D:].reshape(1, D)
    Wg_pad = jnp.zeros((GLANES, D), jnp.float32).at[:E].set(W_g)
    bg_pad = jnp.full((1, GLANES), NEG, jnp.float32).at[0, :E].set(b_g)
    b1r = b1.reshape(E, 1, DH)
    b2r = b2.reshape(E, 1, D)
    full2 = lambda i: (0, 0)
    full3 = lambda i: (0, 0, 0)

    out = pl.pallas_call(
        _moe_body,
        grid=(N_TOK // TILE,),
        in_specs=[
            pl.BlockSpec((TILE, D), lambda i: (i, 0)),
            pl.BlockSpec((D, D), full2),
            pl.BlockSpec((1, D), full2),
            pl.BlockSpec((D, D), full2),
            pl.BlockSpec((1, D), full2),
            pl.BlockSpec((D, D), full2),
            pl.BlockSpec((1, D), full2),
            pl.BlockSpec((GLANES, D), full2),
            pl.BlockSpec((1, GLANES), full2),
            pl.BlockSpec((E, DH, D), full3),
            pl.BlockSpec((E, 1, DH), full3),
            pl.BlockSpec((E, D, DH), full3),
            pl.BlockSpec((E, 1, D), full3),
        ],
        out_specs=pl.BlockSpec((TILE, D), lambda i: (i, 0)),
        out_shape=jax.ShapeDtypeStruct((N_TOK, D), jnp.float32),
    )(x, W_in, b_in.reshape(1, D), Wv, bv, out_proj_w,
      out_proj_b.reshape(1, D), Wg_pad, bg_pad, W1, b1r, W2, b2r)
    return out
